# Initial kernel scaffold; baseline (speedup 1.0000x reference)
import jax
import jax.numpy as jnp
from jax.experimental import pallas as pl


def kernel(boxes, scores):
    raise NotImplementedError("write your pallas kernel here")


# speedup vs baseline: 15.5022x; 15.5022x over previous
import jax
import jax.numpy as jnp
from jax.experimental import pallas as pl

_SCORE_THRESH = 0.25
_NMS_THRESH = 0.5
_MAX_DET = 300
_NEG = -1e9
_NPAD = 5120


def _nms_kernel(x1_ref, y1_ref, x2_ref, y2_ref, sc_ref, out_ref):
    x1 = x1_ref[...]
    y1 = y1_ref[...]
    x2 = x2_ref[...]
    y2 = y2_ref[...]
    sc = sc_ref[...]
    bsz = x1.shape[0]

    work0 = jnp.where(sc > _SCORE_THRESH, sc, _NEG)
    area = (x2 - x1) * (y2 - y1)
    idx = jax.lax.broadcasted_iota(jnp.int32, (bsz, _NPAD), 1)
    lane = jax.lax.broadcasted_iota(jnp.int32, (bsz, 128), 1)

    def step(t, work):
        m = jnp.max(work, axis=1, keepdims=True)
        valid = m > (_NEG * 0.5)
        sel = jnp.where(work == m, idx, jnp.int32(2**30))
        bi = jnp.min(sel, axis=1, keepdims=True)
        bmask = idx == bi
        bx1 = jnp.sum(jnp.where(bmask, x1, 0.0), axis=1, keepdims=True)
        by1 = jnp.sum(jnp.where(bmask, y1, 0.0), axis=1, keepdims=True)
        bx2 = jnp.sum(jnp.where(bmask, x2, 0.0), axis=1, keepdims=True)
        by2 = jnp.sum(jnp.where(bmask, y2, 0.0), axis=1, keepdims=True)
        barea = (bx2 - bx1) * (by2 - by1)
        ix1 = jnp.maximum(bx1, x1)
        iy1 = jnp.maximum(by1, y1)
        ix2 = jnp.minimum(bx2, x2)
        iy2 = jnp.minimum(by2, y2)
        inter = jnp.maximum(ix2 - ix1, 0.0) * jnp.maximum(iy2 - iy1, 0.0)
        iou = inter / (barea + area - inter + 1e-9)
        suppress = (iou > _NMS_THRESH) & valid
        work = jnp.where(suppress | bmask, _NEG, work)

        vf = jnp.where(valid, 1.0, 0.0)
        row = vf * (
            jnp.where(lane == 0, bx1, 0.0)
            + jnp.where(lane == 1, by1, 0.0)
            + jnp.where(lane == 2, bx2, 0.0)
            + jnp.where(lane == 3, by2, 0.0)
            + jnp.where(lane == 4, m, 0.0)
        )
        out_ref[pl.ds(t, 1), :, :] = row[None]
        return work

    jax.lax.fori_loop(0, _MAX_DET, step, work0)


def kernel(boxes, scores):
    bsz, n, _ = boxes.shape
    pad = _NPAD - n
    x1 = jnp.pad(boxes[:, :, 0], ((0, 0), (0, pad)))
    y1 = jnp.pad(boxes[:, :, 1], ((0, 0), (0, pad)))
    x2 = jnp.pad(boxes[:, :, 2], ((0, 0), (0, pad)))
    y2 = jnp.pad(boxes[:, :, 3], ((0, 0), (0, pad)))
    sc = jnp.pad(scores, ((0, 0), (0, pad)), constant_values=-1.0)
    out = pl.pallas_call(
        _nms_kernel,
        out_shape=jax.ShapeDtypeStruct((_MAX_DET, bsz, 128), jnp.float32),
    )(x1, y1, x2, y2, sc)
    return out.transpose(1, 0, 2)[:, :, :5]


# speedup vs baseline: 20.6698x; 1.3333x over previous
import jax
import jax.numpy as jnp
from jax import lax
from jax.experimental import pallas as pl

_SCORE_THRESH = 0.25
_NMS_THRESH = 0.5
_MAX_DET = 300
_NEG = -1e9
_ROWS = 8
_COLS = 640
_NPAD = _ROWS * _COLS


def _nms_kernel(x1_ref, y1_ref, x2_ref, y2_ref, sc_ref, out_ref):
    x1 = x1_ref[...]
    y1 = y1_ref[...]
    x2 = x2_ref[...]
    y2 = y2_ref[...]
    sc = sc_ref[...]
    bsz = x1.shape[0]
    shp = (bsz, _ROWS, _COLS)

    work0 = jnp.where(sc > _SCORE_THRESH, sc, _NEG)
    area = (x2 - x1) * (y2 - y1)
    i1 = lax.broadcasted_iota(jnp.int32, shp, 1)
    i2 = lax.broadcasted_iota(jnp.int32, shp, 2)
    idx = i1 * _COLS + i2
    lane = lax.broadcasted_iota(jnp.int32, (bsz, 128), 1)

    def _red2(x, op):
        return op(op(x, axis=2, keepdims=True), axis=1, keepdims=True)

    def step(t, work):
        m = _red2(work, jnp.max)
        valid = m > (_NEG * 0.5)
        sel = jnp.where(work == m, idx, jnp.int32(2**30))
        bi = _red2(sel, jnp.min)
        bmask = idx == bi
        bx1 = _red2(jnp.where(bmask, x1, 0.0), jnp.sum)
        by1 = _red2(jnp.where(bmask, y1, 0.0), jnp.sum)
        bx2 = _red2(jnp.where(bmask, x2, 0.0), jnp.sum)
        by2 = _red2(jnp.where(bmask, y2, 0.0), jnp.sum)
        barea = (bx2 - bx1) * (by2 - by1)
        ix1 = jnp.maximum(bx1, x1)
        iy1 = jnp.maximum(by1, y1)
        ix2 = jnp.minimum(bx2, x2)
        iy2 = jnp.minimum(by2, y2)
        inter = jnp.maximum(ix2 - ix1, 0.0) * jnp.maximum(iy2 - iy1, 0.0)
        iou = inter / (barea + area - inter + 1e-9)
        suppress = (iou > _NMS_THRESH) & valid
        work = jnp.where(suppress | bmask, _NEG, work)

        vf = jnp.where(valid[:, :, 0], 1.0, 0.0)
        row = vf * (
            jnp.where(lane == 0, bx1[:, :, 0], 0.0)
            + jnp.where(lane == 1, by1[:, :, 0], 0.0)
            + jnp.where(lane == 2, bx2[:, :, 0], 0.0)
            + jnp.where(lane == 3, by2[:, :, 0], 0.0)
            + jnp.where(lane == 4, m[:, :, 0], 0.0)
        )
        out_ref[pl.ds(t, 1), :, :] = row[None]
        return work

    jax.lax.fori_loop(0, _MAX_DET, step, work0)


def kernel(boxes, scores):
    bsz, n, _ = boxes.shape
    pad = _NPAD - n

    def prep(x, cv=0.0):
        return jnp.pad(x, ((0, 0), (0, pad)), constant_values=cv).reshape(
            bsz, _ROWS, _COLS)

    x1 = prep(boxes[:, :, 0])
    y1 = prep(boxes[:, :, 1])
    x2 = prep(boxes[:, :, 2])
    y2 = prep(boxes[:, :, 3])
    sc = prep(scores, -1.0)
    out = pl.pallas_call(
        _nms_kernel,
        out_shape=jax.ShapeDtypeStruct((_MAX_DET, bsz, 128), jnp.float32),
    )(x1, y1, x2, y2, sc)
    return out.transpose(1, 0, 2)[:, :, :5]
